# Initial kernel scaffold; baseline (speedup 1.0000x reference)
import jax
import jax.numpy as jnp
from jax.experimental import pallas as pl


def kernel(z, task_ids, params):
    raise NotImplementedError("write your pallas kernel here")


# speedup vs baseline: 4.2823x; 4.2823x over previous
import functools

import jax
import jax.numpy as jnp
from jax import lax
from jax.experimental import pallas as pl
from jax.experimental.pallas import tpu as pltpu
from jax.experimental.pallas import tpu_sc as plsc

_D = 80
_HE = 64
_SAW = 128
_TILE = 512
_IDXC = 128


def _sc_gather(table, idx):
    _, d = table.shape
    b = idx.shape[0]
    info = plsc.get_sparse_core_info()
    nw = info.num_cores * info.num_subcores
    b_per_w = b // nw
    nchunk = b_per_w // _IDXC
    mesh = plsc.VectorSubcoreMesh(core_axis_name="c", subcore_axis_name="s")

    @functools.partial(
        pl.kernel,
        mesh=mesh,
        out_type=jax.ShapeDtypeStruct((b, d), jnp.float32),
        scratch_types=[
            pltpu.VMEM((nchunk, _IDXC), jnp.int32),
            pltpu.VMEM((nchunk, _IDXC, d), jnp.float32),
            pltpu.SemaphoreType.DMA,
        ],
        compiler_params=pltpu.CompilerParams(use_tc_tiling_on_sc=False),
    )
    def k(table_hbm, idx_hbm, out_hbm, idx_v, rows_v, sem):
        wid = lax.axis_index("s") * info.num_cores + lax.axis_index("c")
        base = wid * b_per_w
        for j in range(nchunk):
            pltpu.sync_copy(idx_hbm.at[pl.ds(base + j * _IDXC, _IDXC)],
                            idx_v.at[j])
        copies = [
            pltpu.async_copy(table_hbm.at[idx_v.at[j]], rows_v.at[j], sem)
            for j in range(nchunk)
        ]
        for c in copies:
            c.wait()
        for j in range(nchunk):
            pltpu.sync_copy(rows_v.at[j],
                            out_hbm.at[pl.ds(base + j * _IDXC, _IDXC)])

    return k(table, idx)


def _tc_body(z_ref, tv_ref,
             wp1_ref, bp1_ref, wp2_ref, bp2_ref, wn3_ref, bn3_ref,
             wa_ref, ba_ref, wb_ref, bb_ref, wc_ref, bc_ref,
             wd_ref, bd_ref, we_ref, be_ref, wf_ref, bf_ref,
             wg_ref, bg_ref, wh_ref, bh_ref, wi_ref, bi_ref,
             hout_ref, cur_ref):
    f32 = jnp.float32

    def mm(x, w_ref, b_ref=None):
        y = jax.lax.dot_general(x, w_ref[...], (((1,), (0,)), ((), ())),
                                preferred_element_type=f32)
        if b_ref is not None:
            y = y + b_ref[...]
        return y

    h = jnp.concatenate([z_ref[...], tv_ref[...]], axis=1)
    t = h.shape[0]
    cur_acc = jnp.zeros((t, 1), f32)
    ctx_acc = jnp.zeros((t, 1), f32)
    for _ in range(2):
        p1 = mm(h, wp1_ref, bp1_ref)
        r = jnp.maximum(p1[:, :208], 0.0)
        s = p1[:, 208:212]
        p2 = jnp.maximum(mm(r, wp2_ref, bp2_ref), 0.0)
        nv2 = p2[:, 160:200]
        novelty = jax.nn.sigmoid(mm(nv2, wn3_ref, bn3_ref))
        d1 = s[:, 0:1] - s[:, 1:2]
        d2 = s[:, 2:3] - s[:, 3:4]
        m0 = (jax.nn.sigmoid(d1) + jax.nn.sigmoid(d2)) * 0.5
        m1 = (jax.nn.sigmoid(-d1) + jax.nn.sigmoid(-d2)) * 0.5
        h_next = p2[:, :80] * m0 + p2[:, 80:160] * m1
        diff = h_next - h
        fe = jnp.sqrt(jnp.sum(diff * diff, axis=1, keepdims=True))
        err = jax.nn.sigmoid(fe - 0.5)
        cur_acc = cur_acc + jnp.clip(0.4 * err + 0.6 * novelty, 0.1, 0.9)
        ctx_acc = ctx_acc + (m0 + m1)
        h = h_next
    ctx = ctx_acc * 0.25

    w = jnp.maximum(mm(jnp.concatenate([h, ctx * h], axis=1), wa_ref, ba_ref), 0.0)
    w = jnp.maximum(mm(w, wb_ref, bb_ref), 0.0)
    why = mm(w, wc_ref, bc_ref)
    qt = jnp.maximum(mm(jnp.concatenate([h, why], axis=1), wd_ref, bd_ref), 0.0)
    qt = jnp.maximum(mm(qt, we_ref, be_ref), 0.0)
    f = mm(qt, wf_ref, bf_ref)
    if_q = jax.nn.sigmoid(f[:, 0:1])
    blended = h + if_q * f[:, 1:81]
    o = jnp.maximum(mm(jnp.concatenate([h, blended], axis=1), wg_ref, bg_ref), 0.0)
    o = jnp.maximum(mm(o, wh_ref, bh_ref), 0.0)
    h_ref_out = mm(o, wi_ref, bi_ref)
    hout_ref[...] = 0.7 * h + 0.3 * h_ref_out
    cur_ref[...] = cur_acc * 0.5


def _tc_forward(z, tv, weights):
    b = z.shape[0]
    grid = (b // _TILE,)

    def rowspec(cols):
        return pl.BlockSpec((_TILE, cols), lambda i: (i, 0))

    def fullspec(arr):
        return pl.BlockSpec(arr.shape, lambda i: (0,) * arr.ndim)

    in_specs = [rowspec(z.shape[1]), rowspec(tv.shape[1])]
    in_specs += [fullspec(wgt) for wgt in weights]
    return pl.pallas_call(
        _tc_body,
        grid=grid,
        in_specs=in_specs,
        out_specs=[rowspec(_D), rowspec(1)],
        out_shape=[
            jax.ShapeDtypeStruct((b, _D), jnp.float32),
            jax.ShapeDtypeStruct((b, 1), jnp.float32),
        ],
        compiler_params=pltpu.CompilerParams(
            dimension_semantics=("parallel",),
        ),
    )(z, tv, *weights)


def _prep_weights(p):
    f32 = jnp.float32
    zeros = jnp.zeros

    def row(x):
        return x.reshape(1, -1).astype(f32)

    e1 = p['eW1'].reshape(2 * _HE, _D)
    wp1 = jnp.concatenate([e1, p['Wn1'], p['Wr1'][:2], p['Wr2'][:2]], axis=0)
    bp1 = jnp.concatenate([p['eb1'].reshape(-1), p['bn1'],
                           p['br1'][:2], p['br2'][:2]])
    top = jnp.concatenate([p['eW2'][0], zeros((_D, _HE), f32),
                           zeros((_D, _D), f32)], axis=1)
    mid = jnp.concatenate([zeros((_D, _HE), f32), p['eW2'][1],
                           zeros((_D, _D), f32)], axis=1)
    bot = jnp.concatenate([zeros((_D // 2, 2 * _HE), f32), p['Wn2']], axis=1)
    wp2 = jnp.concatenate([top, mid, bot], axis=0)
    bp2 = jnp.concatenate([p['eb2'].reshape(-1), p['bn2']])

    wd = jnp.concatenate([p['Wi1'], p['Wt1']], axis=0)
    bd = jnp.concatenate([p['bi1'], p['bt1']])
    we = jnp.concatenate([
        jnp.concatenate([p['Wi2'], zeros((_HE, _SAW), f32)], axis=1),
        jnp.concatenate([zeros((_SAW, _SAW), f32), p['Wt2']], axis=1),
    ], axis=0)
    be = jnp.concatenate([p['bi2'], p['bt2']])
    wf = jnp.concatenate([
        jnp.concatenate([p['Wi3'], zeros((1, _SAW), f32)], axis=1),
        jnp.concatenate([zeros((_D, _HE), f32), p['Wt3']], axis=1),
    ], axis=0)
    bf = jnp.concatenate([p['bi3'], p['bt3']])

    return [
        wp1.T, row(bp1), wp2.T, row(bp2), p['Wn3'].T, row(p['bn3']),
        p['Ww1'].T, row(p['bw1']), p['Ww2'].T, row(p['bw2']),
        p['Ww3'].T, row(p['bw3']),
        wd.T, row(bd), we.T, row(be), wf.T, row(bf),
        p['Wo1'].T, row(p['bo1']), p['Wo2'].T, row(p['bo2']),
        p['Wo3'].T, row(p['bo3']),
    ]


def kernel(z, task_ids, params):
    tv = _sc_gather(params['task_emb'].astype(jnp.float32),
                    task_ids.astype(jnp.int32))
    weights = _prep_weights(params)
    h_final, avg_cur = _tc_forward(z.astype(jnp.float32), tv, weights)
    return h_final, avg_cur


# speedup vs baseline: 6.8665x; 1.6035x over previous
import functools

import jax
import jax.numpy as jnp
from jax import lax
from jax.experimental import pallas as pl
from jax.experimental.pallas import tpu as pltpu
from jax.experimental.pallas import tpu_sc as plsc

_D = 80
_HE = 64
_SAW = 128
_TILE = 2048
_NCHAIN = 1
_IDXC = 128


def _sc_gather(table, idx):
    _, d = table.shape
    b = idx.shape[0]
    info = plsc.get_sparse_core_info()
    nw = info.num_cores * info.num_subcores
    b_per_w = b // nw
    nchunk = b_per_w // _IDXC
    mesh = plsc.VectorSubcoreMesh(core_axis_name="c", subcore_axis_name="s")

    @functools.partial(
        pl.kernel,
        mesh=mesh,
        out_type=jax.ShapeDtypeStruct((b, d), jnp.float32),
        scratch_types=[
            pltpu.VMEM((nchunk, _IDXC), jnp.int32),
            pltpu.VMEM((nchunk, _IDXC, d), jnp.float32),
            pltpu.SemaphoreType.DMA,
        ],
        compiler_params=pltpu.CompilerParams(use_tc_tiling_on_sc=False),
    )
    def k(table_hbm, idx_hbm, out_hbm, idx_v, rows_v, sem):
        wid = lax.axis_index("s") * info.num_cores + lax.axis_index("c")
        base = wid * b_per_w
        for j in range(nchunk):
            pltpu.sync_copy(idx_hbm.at[pl.ds(base + j * _IDXC, _IDXC)],
                            idx_v.at[j])
        copies = [
            pltpu.async_copy(table_hbm.at[idx_v.at[j]], rows_v.at[j], sem)
            for j in range(nchunk)
        ]
        for c in copies:
            c.wait()
        for j in range(nchunk):
            pltpu.sync_copy(rows_v.at[j],
                            out_hbm.at[pl.ds(base + j * _IDXC, _IDXC)])

    return k(table, idx)


def _tc_body(z_ref, tv_ref,
             wp1_ref, wp2_ref, wn3_ref,
             wa_ref, wb_ref, wc_ref, wd_ref, we_ref, wf_ref,
             wg_ref, wh_ref, wi_ref,
             hout_ref, cur_ref):
    f32 = jnp.float32

    def mm(w_ref, x):
        y = lax.dot_general(w_ref[...][:, :-1], x, (((1,), (0,)), ((), ())),
                            preferred_element_type=f32)
        return y + w_ref[...][:, -1:]

    sig = jax.nn.sigmoid

    t = z_ref.shape[0]
    n = t // _NCHAIN
    cols = [pl.ds(c * n, n) for c in range(_NCHAIN)]

    def smap(f, *lists):
        return [f(*args) for args in zip(*lists)]

    hs = smap(lambda rs: jnp.concatenate(
        [jnp.transpose(z_ref[rs, :]), jnp.transpose(tv_ref[rs, :])], axis=0),
        cols)
    cur_accs = [jnp.zeros((1, n), f32) for _ in cols]
    ctx_accs = [jnp.zeros((1, n), f32) for _ in cols]
    for _ in range(2):
        p1s = smap(lambda h: mm(wp1_ref, h), hs)
        rrs = smap(lambda p1: jnp.maximum(p1[:208], 0.0), p1s)
        p2s = smap(lambda r: jnp.maximum(mm(wp2_ref, r), 0.0), rrs)
        novs = smap(lambda p2: sig(mm(wn3_ref, p2[160:200])), p2s)

        def mixes(p1):
            d1 = p1[208:209] - p1[209:210]
            d2 = p1[210:211] - p1[211:212]
            return ((sig(d1) + sig(d2)) * 0.5, (sig(-d1) + sig(-d2)) * 0.5)
        ms = smap(mixes, p1s)
        hns = smap(lambda p2, m: p2[:80] * m[0] + p2[80:160] * m[1], p2s, ms)

        def curiosity(h, h_next, novelty):
            diff = h_next - h
            fe = jnp.sqrt(jnp.sum(diff * diff, axis=0, keepdims=True))
            return jnp.clip(0.4 * sig(fe - 0.5) + 0.6 * novelty, 0.1, 0.9)
        cur_accs = smap(lambda a, h, hn, nov: a + curiosity(h, hn, nov),
                        cur_accs, hs, hns, novs)
        ctx_accs = smap(lambda a, m: a + m[0] + m[1], ctx_accs, ms)
        hs = hns
    ctxs = [a * 0.25 for a in ctx_accs]

    ws = smap(lambda h, ctx: jnp.maximum(
        mm(wa_ref, jnp.concatenate([h, ctx * h], axis=0)), 0.0), hs, ctxs)
    ws = smap(lambda w: jnp.maximum(mm(wb_ref, w), 0.0), ws)
    whys = smap(lambda w: mm(wc_ref, w), ws)
    qts = smap(lambda h, why: jnp.maximum(
        mm(wd_ref, jnp.concatenate([h, why], axis=0)), 0.0), hs, whys)
    qts = smap(lambda qt: jnp.maximum(mm(we_ref, qt), 0.0), qts)
    fs = smap(lambda qt: mm(wf_ref, qt), qts)
    blends = smap(lambda h, f: h + sig(f[80:81]) * f[:80], hs, fs)
    os_ = smap(lambda h, bl: jnp.maximum(
        mm(wg_ref, jnp.concatenate([h, bl], axis=0)), 0.0), hs, blends)
    os_ = smap(lambda o: jnp.maximum(mm(wh_ref, o), 0.0), os_)
    hfs = smap(lambda h, o: 0.7 * h + 0.3 * mm(wi_ref, o), hs, os_)
    for c, rs in enumerate(cols):
        hout_ref[rs, :] = jnp.transpose(hfs[c])
        cur_ref[rs, :] = jnp.transpose(cur_accs[c] * 0.5)


def _tc_forward(z, tv, weights):
    b = z.shape[0]
    grid = (b // _TILE,)

    def rowspec(cols):
        return pl.BlockSpec((_TILE, cols), lambda i: (i, 0))

    def fullspec(arr):
        return pl.BlockSpec(arr.shape, lambda i: (0,) * arr.ndim)

    in_specs = [rowspec(z.shape[1]), rowspec(tv.shape[1])]
    in_specs += [fullspec(wgt) for wgt in weights]
    return pl.pallas_call(
        _tc_body,
        grid=grid,
        in_specs=in_specs,
        out_specs=[rowspec(_D), rowspec(1)],
        out_shape=[
            jax.ShapeDtypeStruct((b, _D), jnp.float32),
            jax.ShapeDtypeStruct((b, 1), jnp.float32),
        ],
        compiler_params=pltpu.CompilerParams(
            dimension_semantics=("parallel",),
        ),
    )(z, tv, *weights)


def _prep_weights(p):
    f32 = jnp.float32
    zeros = jnp.zeros

    def wb(w, b):
        return jnp.concatenate([w, b.reshape(-1, 1)], axis=1).astype(f32)

    e1 = p['eW1'].reshape(2 * _HE, _D)
    wp1 = wb(jnp.concatenate([e1, p['Wn1'], p['Wr1'][:2], p['Wr2'][:2]], 0),
             jnp.concatenate([p['eb1'].reshape(-1), p['bn1'],
                              p['br1'][:2], p['br2'][:2]]))
    top = jnp.concatenate([p['eW2'][0], zeros((_D, _HE), f32),
                           zeros((_D, _D), f32)], axis=1)
    mid = jnp.concatenate([zeros((_D, _HE), f32), p['eW2'][1],
                           zeros((_D, _D), f32)], axis=1)
    bot = jnp.concatenate([zeros((_D // 2, 2 * _HE), f32), p['Wn2']], axis=1)
    wp2 = wb(jnp.concatenate([top, mid, bot], axis=0),
             jnp.concatenate([p['eb2'].reshape(-1), p['bn2']]))
    wn3 = wb(p['Wn3'], p['bn3'])

    wd = wb(jnp.concatenate([p['Wi1'], p['Wt1']], axis=0),
            jnp.concatenate([p['bi1'], p['bt1']]))
    we = wb(jnp.concatenate([
        jnp.concatenate([p['Wi2'], zeros((_HE, _SAW), f32)], axis=1),
        jnp.concatenate([zeros((_SAW, _SAW), f32), p['Wt2']], axis=1),
    ], axis=0), jnp.concatenate([p['bi2'], p['bt2']]))
    wf = wb(jnp.concatenate([
        jnp.concatenate([zeros((_D, _HE), f32), p['Wt3']], axis=1),
        jnp.concatenate([p['Wi3'], zeros((1, _SAW), f32)], axis=1),
    ], axis=0), jnp.concatenate([p['bt3'], p['bi3']]))

    return [
        wp1, wp2, wn3,
        wb(p['Ww1'], p['bw1']), wb(p['Ww2'], p['bw2']), wb(p['Ww3'], p['bw3']),
        wd, we, wf,
        wb(p['Wo1'], p['bo1']), wb(p['Wo2'], p['bo2']), wb(p['Wo3'], p['bo3']),
    ]


def kernel(z, task_ids, params):
    tv = _sc_gather(params['task_emb'].astype(jnp.float32),
                    task_ids.astype(jnp.int32))
    weights = _prep_weights(params)
    h_final, avg_cur = _tc_forward(z.astype(jnp.float32), tv, weights)
    return h_final, avg_cur


# speedup vs baseline: 8.2361x; 1.1995x over previous
import functools

import jax
import jax.numpy as jnp
from jax import lax
from jax.experimental import pallas as pl
from jax.experimental.pallas import tpu as pltpu
from jax.experimental.pallas import tpu_sc as plsc

_D = 80
_HE = 64
_SAW = 128
_TILE = 2048
_IDXC = 128


def _sc_gather(table, idx):
    _, d = table.shape
    b = idx.shape[0]
    info = plsc.get_sparse_core_info()
    nw = info.num_cores * info.num_subcores
    b_per_w = b // nw
    nchunk = b_per_w // _IDXC
    mesh = plsc.VectorSubcoreMesh(core_axis_name="c", subcore_axis_name="s")

    @functools.partial(
        pl.kernel,
        mesh=mesh,
        out_type=jax.ShapeDtypeStruct((b, d), jnp.float32),
        scratch_types=[
            pltpu.VMEM((nchunk, _IDXC), jnp.int32),
            pltpu.VMEM((nchunk, _IDXC, d), jnp.float32),
            pltpu.SemaphoreType.DMA,
        ],
        compiler_params=pltpu.CompilerParams(use_tc_tiling_on_sc=False),
    )
    def k(table_hbm, idx_hbm, out_hbm, idx_v, rows_v, sem):
        wid = lax.axis_index("s") * info.num_cores + lax.axis_index("c")
        base = wid * b_per_w
        for j in range(nchunk):
            pltpu.sync_copy(idx_hbm.at[pl.ds(base + j * _IDXC, _IDXC)],
                            idx_v.at[j])
        copies = [
            pltpu.async_copy(table_hbm.at[idx_v.at[j]], rows_v.at[j], sem)
            for j in range(nchunk)
        ]
        for c in copies:
            c.wait()
        for j in range(nchunk):
            pltpu.sync_copy(rows_v.at[j],
                            out_hbm.at[pl.ds(base + j * _IDXC, _IDXC)])

    return k(table, idx)


def _tc_body(z_ref, tv_ref, ball_ref,
             ew1_ref, wn1_ref, wr1_ref, wr2_ref, ew2_ref, wn2_ref, wn3_ref,
             ww1_ref, ww2_ref, ww3_ref, wi1_ref, wt1_ref, wi2_ref, wt2_ref,
             wi3_ref, wt3_ref, wo1_ref, wo2_ref, wo3_ref,
             hout_ref, cur_ref,
             wp1_s, wp2_s, wd_s, we_s, wf_s,
             bp1_s, bp2_s, bn3_s, ba_s, bb_s, bc_s, bd_s, be_s, bf_s,
             bg_s, bh_s, bi_s):
    f32 = jnp.float32

    @pl.when(pl.program_id(0) == 0)
    def _prep():
        bcol = jnp.transpose(ball_ref[...])

        wp1_s[0:64, :] = ew1_ref[0]
        wp1_s[64:128, :] = ew1_ref[1]
        wp1_s[128:208, :] = wn1_ref[...]
        wp1_s[208:210, :] = wr1_ref[0:2, :]
        wp1_s[210:212, :] = wr2_ref[0:2, :]
        bp1_s[0:128, :] = bcol[0:128]
        bp1_s[128:208, :] = bcol[128:208]
        bp1_s[208:210, :] = bcol[208:210]
        bp1_s[210:212, :] = bcol[272:274]

        wp2_s[...] = jnp.zeros(wp2_s.shape, f32)
        wp2_s[0:80, 0:64] = ew2_ref[0]
        wp2_s[80:160, 64:128] = ew2_ref[1]
        wp2_s[160:200, 128:208] = wn2_ref[...]
        bp2_s[0:160, :] = bcol[336:496]
        bp2_s[160:200, :] = bcol[496:536]
        bn3_s[...] = bcol[1720:1721]

        ba_s[...] = bcol[536:664]
        bb_s[...] = bcol[664:792]
        bc_s[...] = bcol[792:856]

        wd_s[0:128, :] = wi1_ref[...]
        wd_s[128:256, :] = wt1_ref[...]
        bd_s[0:128, :] = bcol[856:984]
        bd_s[128:256, :] = bcol[984:1112]

        we_s[...] = jnp.zeros(we_s.shape, f32)
        we_s[0:64, 0:128] = wi2_ref[...]
        we_s[64:192, 128:256] = wt2_ref[...]
        be_s[0:64, :] = bcol[1112:1176]
        be_s[64:192, :] = bcol[1176:1304]

        wf_s[...] = jnp.zeros(wf_s.shape, f32)
        wf_s[0:80, 64:192] = wt3_ref[...]
        wf_s[80:81, 0:64] = wi3_ref[...]
        bf_s[0:80, :] = bcol[1304:1384]
        bf_s[80:81, :] = bcol[1721:1722]

        bg_s[...] = bcol[1384:1512]
        bh_s[...] = bcol[1512:1640]
        bi_s[...] = bcol[1640:1720]

    def mm(w, b_ref, x):
        y = lax.dot_general(w, x, (((1,), (0,)), ((), ())),
                            preferred_element_type=f32)
        return y + b_ref[...]

    sig = jax.nn.sigmoid
    n = z_ref.shape[0]
    h = jnp.concatenate(
        [jnp.transpose(z_ref[...]), jnp.transpose(tv_ref[...])], axis=0)
    cur_acc = jnp.zeros((1, n), f32)
    ctx_acc = jnp.zeros((1, n), f32)
    for _ in range(2):
        p1 = mm(wp1_s[...], bp1_s, h)
        r = jnp.maximum(p1[:208], 0.0)
        p2 = jnp.maximum(mm(wp2_s[...], bp2_s, r), 0.0)
        novelty = sig(mm(wn3_ref[...], bn3_s, p2[160:200]))
        d1 = p1[208:209] - p1[209:210]
        d2 = p1[210:211] - p1[211:212]
        m0 = (sig(d1) + sig(d2)) * 0.5
        m1 = (sig(-d1) + sig(-d2)) * 0.5
        h_next = p2[:80] * m0 + p2[80:160] * m1
        diff = h_next - h
        fe = jnp.sqrt(jnp.sum(diff * diff, axis=0, keepdims=True))
        err = sig(fe - 0.5)
        cur_acc = cur_acc + jnp.clip(0.4 * err + 0.6 * novelty, 0.1, 0.9)
        ctx_acc = ctx_acc + (m0 + m1)
        h = h_next
    ctx = ctx_acc * 0.25

    w = jnp.maximum(mm(ww1_ref[...], ba_s,
                       jnp.concatenate([h, ctx * h], axis=0)), 0.0)
    w = jnp.maximum(mm(ww2_ref[...], bb_s, w), 0.0)
    why = mm(ww3_ref[...], bc_s, w)
    qt = jnp.maximum(mm(wd_s[...], bd_s,
                        jnp.concatenate([h, why], axis=0)), 0.0)
    qt = jnp.maximum(mm(we_s[...], be_s, qt), 0.0)
    f = mm(wf_s[...], bf_s, qt)
    blended = h + sig(f[80:81]) * f[:80]
    o = jnp.maximum(mm(wo1_ref[...], bg_s,
                       jnp.concatenate([h, blended], axis=0)), 0.0)
    o = jnp.maximum(mm(wo2_ref[...], bh_s, o), 0.0)
    h_final = 0.7 * h + 0.3 * mm(wo3_ref[...], bi_s, o)
    hout_ref[...] = jnp.transpose(h_final)
    cur_ref[...] = jnp.transpose(cur_acc * 0.5)


def _tc_forward(z, tv, ball, raw):
    b = z.shape[0]
    grid = (b // _TILE,)
    f32 = jnp.float32

    def rowspec(cols):
        return pl.BlockSpec((_TILE, cols), lambda i: (i, 0))

    def fullspec(arr):
        return pl.BlockSpec(arr.shape, lambda i: (0,) * arr.ndim)

    in_specs = [rowspec(z.shape[1]), rowspec(tv.shape[1]), fullspec(ball)]
    in_specs += [fullspec(x) for x in raw]
    return pl.pallas_call(
        _tc_body,
        grid=grid,
        in_specs=in_specs,
        out_specs=[rowspec(_D), rowspec(1)],
        out_shape=[
            jax.ShapeDtypeStruct((b, _D), f32),
            jax.ShapeDtypeStruct((b, 1), f32),
        ],
        scratch_shapes=[
            pltpu.VMEM((212, 80), f32),
            pltpu.VMEM((200, 208), f32),
            pltpu.VMEM((256, 144), f32),
            pltpu.VMEM((192, 256), f32),
            pltpu.VMEM((81, 192), f32),
            pltpu.VMEM((212, 1), f32), pltpu.VMEM((200, 1), f32),
            pltpu.VMEM((1, 1), f32), pltpu.VMEM((128, 1), f32),
            pltpu.VMEM((128, 1), f32), pltpu.VMEM((64, 1), f32),
            pltpu.VMEM((256, 1), f32), pltpu.VMEM((192, 1), f32),
            pltpu.VMEM((81, 1), f32), pltpu.VMEM((128, 1), f32),
            pltpu.VMEM((128, 1), f32), pltpu.VMEM((80, 1), f32),
        ],
        compiler_params=pltpu.CompilerParams(
            dimension_semantics=("arbitrary",),
        ),
    )(z, tv, ball, *raw)


def kernel(z, task_ids, params):
    p = params
    tv = _sc_gather(p['task_emb'].astype(jnp.float32),
                    task_ids.astype(jnp.int32))
    ball = jnp.concatenate(
        [p[k].reshape(1, -1).astype(jnp.float32) for k in
         ('eb1', 'bn1', 'br1', 'br2', 'eb2', 'bn2', 'bw1', 'bw2', 'bw3',
          'bi1', 'bt1', 'bi2', 'bt2', 'bt3', 'bo1', 'bo2', 'bo3', 'bn3',
          'bi3')], axis=1)
    raw = [p[k] for k in
           ('eW1', 'Wn1', 'Wr1', 'Wr2', 'eW2', 'Wn2', 'Wn3', 'Ww1', 'Ww2',
            'Ww3', 'Wi1', 'Wt1', 'Wi2', 'Wt2', 'Wi3', 'Wt3', 'Wo1', 'Wo2',
            'Wo3')]
    h_final, avg_cur = _tc_forward(z, tv, ball, raw)
    return h_final, avg_cur


# speedup vs baseline: 8.5343x; 1.0362x over previous
import functools

import jax
import jax.numpy as jnp
from jax import lax
from jax.experimental import pallas as pl
from jax.experimental.pallas import tpu as pltpu
from jax.experimental.pallas import tpu_sc as plsc

_D = 80
_HE = 64
_SAW = 128
_TILE = 4096
_IDXC = 128


def _sc_gather(table, idx):
    _, d = table.shape
    b = idx.shape[0]
    info = plsc.get_sparse_core_info()
    nw = info.num_cores * info.num_subcores
    b_per_w = b // nw
    nchunk = b_per_w // _IDXC
    mesh = plsc.VectorSubcoreMesh(core_axis_name="c", subcore_axis_name="s")

    @functools.partial(
        pl.kernel,
        mesh=mesh,
        out_type=jax.ShapeDtypeStruct((b, d), jnp.float32),
        scratch_types=[
            pltpu.VMEM((nchunk, _IDXC), jnp.int32),
            pltpu.VMEM((nchunk, _IDXC, d), jnp.float32),
            pltpu.SemaphoreType.DMA,
        ],
        compiler_params=pltpu.CompilerParams(use_tc_tiling_on_sc=False),
    )
    def k(table_hbm, idx_hbm, out_hbm, idx_v, rows_v, sem):
        wid = lax.axis_index("s") * info.num_cores + lax.axis_index("c")
        base = wid * b_per_w
        for j in range(nchunk):
            pltpu.sync_copy(idx_hbm.at[pl.ds(base + j * _IDXC, _IDXC)],
                            idx_v.at[j])
        copies = [
            pltpu.async_copy(table_hbm.at[idx_v.at[j]], rows_v.at[j], sem)
            for j in range(nchunk)
        ]
        for c in copies:
            c.wait()
        for j in range(nchunk):
            pltpu.sync_copy(rows_v.at[j],
                            out_hbm.at[pl.ds(base + j * _IDXC, _IDXC)])

    return k(table, idx)


def _tc_body(z_ref, tv_ref, ball_ref,
             ew1_ref, wn1_ref, wr1_ref, wr2_ref, ew2_ref, wn2_ref, wn3_ref,
             ww1_ref, ww2_ref, ww3_ref, wi1_ref, wt1_ref, wi2_ref, wt2_ref,
             wi3_ref, wt3_ref, wo1_ref, wo2_ref, wo3_ref,
             hout_ref, cur_ref,
             wp1_s, wp2_s, wd_s, we_s, wf_s,
             bp1_s, bp2_s, bn3_s, ba_s, bb_s, bc_s, bd_s, be_s, bf_s,
             bg_s, bh_s, bi_s):
    f32 = jnp.float32

    @pl.when(pl.program_id(0) == 0)
    def _prep():
        bcol = jnp.transpose(ball_ref[...])

        wp1_s[0:64, :] = ew1_ref[0]
        wp1_s[64:128, :] = ew1_ref[1]
        wp1_s[128:208, :] = wn1_ref[...]
        wp1_s[208:210, :] = wr1_ref[0:2, :]
        wp1_s[210:212, :] = wr2_ref[0:2, :]
        bp1_s[0:128, :] = bcol[0:128]
        bp1_s[128:208, :] = bcol[128:208]
        bp1_s[208:210, :] = bcol[208:210]
        bp1_s[210:212, :] = bcol[272:274]

        wp2_s[...] = jnp.zeros(wp2_s.shape, f32)
        wp2_s[0:80, 0:64] = ew2_ref[0]
        wp2_s[80:160, 64:128] = ew2_ref[1]
        wp2_s[160:200, 128:208] = wn2_ref[...]
        bp2_s[0:160, :] = bcol[336:496]
        bp2_s[160:200, :] = bcol[496:536]
        bn3_s[...] = bcol[1720:1721]

        ba_s[...] = bcol[536:664]
        bb_s[...] = bcol[664:792]
        bc_s[...] = bcol[792:856]

        wd_s[0:128, :] = wi1_ref[...]
        wd_s[128:256, :] = wt1_ref[...]
        bd_s[0:128, :] = bcol[856:984]
        bd_s[128:256, :] = bcol[984:1112]

        we_s[...] = jnp.zeros(we_s.shape, f32)
        we_s[0:64, 0:128] = wi2_ref[...]
        we_s[64:192, 128:256] = wt2_ref[...]
        be_s[0:64, :] = bcol[1112:1176]
        be_s[64:192, :] = bcol[1176:1304]

        wf_s[...] = jnp.zeros(wf_s.shape, f32)
        wf_s[0:80, 64:192] = wt3_ref[...]
        wf_s[80:81, 0:64] = wi3_ref[...]
        bf_s[0:80, :] = bcol[1304:1384]
        bf_s[80:81, :] = bcol[1721:1722]

        bg_s[...] = bcol[1384:1512]
        bh_s[...] = bcol[1512:1640]
        bi_s[...] = bcol[1640:1720]

    def mm(w, b_ref, x):
        y = lax.dot_general(w, x, (((1,), (0,)), ((), ())),
                            preferred_element_type=f32)
        return y + b_ref[...]

    sig = jax.nn.sigmoid
    n = z_ref.shape[0]
    h = jnp.concatenate(
        [jnp.transpose(z_ref[...]), jnp.transpose(tv_ref[...])], axis=0)
    cur_acc = jnp.zeros((1, n), f32)
    ctx_acc = jnp.zeros((1, n), f32)
    for _ in range(2):
        p1 = mm(wp1_s[...], bp1_s, h)
        r = jnp.maximum(p1[:208], 0.0)
        p2 = jnp.maximum(mm(wp2_s[...], bp2_s, r), 0.0)
        novelty = sig(mm(wn3_ref[...], bn3_s, p2[160:200]))
        d1 = p1[208:209] - p1[209:210]
        d2 = p1[210:211] - p1[211:212]
        m0 = (sig(d1) + sig(d2)) * 0.5
        m1 = (sig(-d1) + sig(-d2)) * 0.5
        h_next = p2[:80] * m0 + p2[80:160] * m1
        diff = h_next - h
        fe = jnp.sqrt(jnp.sum(diff * diff, axis=0, keepdims=True))
        err = sig(fe - 0.5)
        cur_acc = cur_acc + jnp.clip(0.4 * err + 0.6 * novelty, 0.1, 0.9)
        ctx_acc = ctx_acc + (m0 + m1)
        h = h_next
    ctx = ctx_acc * 0.25

    w = jnp.maximum(mm(ww1_ref[...], ba_s,
                       jnp.concatenate([h, ctx * h], axis=0)), 0.0)
    w = jnp.maximum(mm(ww2_ref[...], bb_s, w), 0.0)
    why = mm(ww3_ref[...], bc_s, w)
    qt = jnp.maximum(mm(wd_s[...], bd_s,
                        jnp.concatenate([h, why], axis=0)), 0.0)
    qt = jnp.maximum(mm(we_s[...], be_s, qt), 0.0)
    f = mm(wf_s[...], bf_s, qt)
    blended = h + sig(f[80:81]) * f[:80]
    o = jnp.maximum(mm(wo1_ref[...], bg_s,
                       jnp.concatenate([h, blended], axis=0)), 0.0)
    o = jnp.maximum(mm(wo2_ref[...], bh_s, o), 0.0)
    h_final = 0.7 * h + 0.3 * mm(wo3_ref[...], bi_s, o)
    hout_ref[...] = jnp.transpose(h_final)
    cur_ref[...] = jnp.transpose(cur_acc * 0.5)


def _tc_forward(z, tv, ball, raw):
    b = z.shape[0]
    grid = (b // _TILE,)
    f32 = jnp.float32

    def rowspec(cols):
        return pl.BlockSpec((_TILE, cols), lambda i: (i, 0))

    def fullspec(arr):
        return pl.BlockSpec(arr.shape, lambda i: (0,) * arr.ndim)

    in_specs = [rowspec(z.shape[1]), rowspec(tv.shape[1]), fullspec(ball)]
    in_specs += [fullspec(x) for x in raw]
    return pl.pallas_call(
        _tc_body,
        grid=grid,
        in_specs=in_specs,
        out_specs=[rowspec(_D), rowspec(1)],
        out_shape=[
            jax.ShapeDtypeStruct((b, _D), f32),
            jax.ShapeDtypeStruct((b, 1), f32),
        ],
        scratch_shapes=[
            pltpu.VMEM((212, 80), f32),
            pltpu.VMEM((200, 208), f32),
            pltpu.VMEM((256, 144), f32),
            pltpu.VMEM((192, 256), f32),
            pltpu.VMEM((81, 192), f32),
            pltpu.VMEM((212, 1), f32), pltpu.VMEM((200, 1), f32),
            pltpu.VMEM((1, 1), f32), pltpu.VMEM((128, 1), f32),
            pltpu.VMEM((128, 1), f32), pltpu.VMEM((64, 1), f32),
            pltpu.VMEM((256, 1), f32), pltpu.VMEM((192, 1), f32),
            pltpu.VMEM((81, 1), f32), pltpu.VMEM((128, 1), f32),
            pltpu.VMEM((128, 1), f32), pltpu.VMEM((80, 1), f32),
        ],
        compiler_params=pltpu.CompilerParams(
            dimension_semantics=("arbitrary",),
        ),
    )(z, tv, ball, *raw)


def kernel(z, task_ids, params):
    p = params
    tv = _sc_gather(p['task_emb'].astype(jnp.float32),
                    task_ids.astype(jnp.int32))
    ball = jnp.concatenate(
        [p[k].reshape(1, -1).astype(jnp.float32) for k in
         ('eb1', 'bn1', 'br1', 'br2', 'eb2', 'bn2', 'bw1', 'bw2', 'bw3',
          'bi1', 'bt1', 'bi2', 'bt2', 'bt3', 'bo1', 'bo2', 'bo3', 'bn3',
          'bi3')], axis=1)
    raw = [p[k] for k in
           ('eW1', 'Wn1', 'Wr1', 'Wr2', 'eW2', 'Wn2', 'Wn3', 'Ww1', 'Ww2',
            'Ww3', 'Wi1', 'Wt1', 'Wi2', 'Wt2', 'Wi3', 'Wt3', 'Wo1', 'Wo2',
            'Wo3')]
    h_final, avg_cur = _tc_forward(z, tv, ball, raw)
    return h_final, avg_cur
